# baseline (speedup 1.0000x reference)
import functools

import jax
import jax.numpy as jnp
from jax import lax
from jax.experimental import pallas as pl
from jax.experimental.pallas import tpu as pltpu
from jax.experimental.pallas import tpu_sc as plsc

NPOINT = 512
RADIUS = 0.2
NSAMPLE = 32
B = 16
N = 4096
EPS = 1e-5

NC, NS, L = 2, 16, 16
NW = NC * NS
ROWS_PER_W = B * NPOINT // NW


def _vec_take(v, idx):
    return lax.gather(
        v, idx[:, None],
        lax.GatherDimensionNumbers(offset_dims=(), collapsed_slice_dims=(0,),
                                   start_index_map=(0,)),
        slice_sizes=(1,),
        mode=lax.GatherScatterMode.PROMISE_IN_BOUNDS)


def _rtne_bf16(v):
    u = lax.bitcast_convert_type(v, jnp.uint32)
    u2 = u + jnp.uint32(0x7FFF) + ((u >> 16) & jnp.uint32(1))
    return lax.bitcast_convert_type(u2 & jnp.uint32(0xFFFF0000), jnp.float32)


def _fps_kernel(xyz_ref, out_ref, xyzb_ref, newb_ref, dist_ref):
    x = xyz_ref[:, 0, :]
    y = xyz_ref[:, 1, :]
    z = xyz_ref[:, 2, :]
    p48 = jnp.concatenate([x, y, z], axis=0)
    iota = lax.broadcasted_iota(jnp.int32, (B, N), 1)
    siota3 = lax.broadcasted_iota(jnp.int32, (3, B, NPOINT), 2)
    big = jnp.int32(N)

    m0 = jnp.max(x, axis=1, keepdims=True)
    far0 = jnp.min(jnp.where(x == m0, iota, big), axis=1, keepdims=True)
    dist_ref[...] = jnp.full((B, N), 1e10, dtype=jnp.float32)
    acc0 = jnp.zeros((3, B, NPOINT), jnp.float32)

    def body(i, carry):
        far, acc3 = carry
        onehot = iota == far
        oh3 = jnp.concatenate([onehot, onehot, onehot], axis=0)
        ext = jnp.sum(jnp.where(oh3, p48, 0.0), axis=1, keepdims=True)
        acc3 = jnp.where(siota3 == i, ext.reshape(3, B, 1), acc3)
        diff = p48 - ext
        sq = diff * diff
        d = (sq[0:B] + sq[B:2 * B]) + sq[2 * B:3 * B]
        dist = dist_ref[...]
        dist = jnp.where(d < dist, d, dist)
        dist_ref[...] = dist
        m = jnp.max(dist, axis=1, keepdims=True)
        far = jnp.min(jnp.where(dist == m, iota, big), axis=1, keepdims=True)
        return far, acc3

    _, acc3 = lax.fori_loop(0, NPOINT, body, (far0, acc0))
    out_ref[...] = acc3.transpose(1, 0, 2)

    tn = (x * x + y * y) + z * z
    xyzb_ref[...] = jnp.stack([_rtne_bf16(x), _rtne_bf16(y), _rtne_bf16(z), tn],
                              axis=1)
    cx = acc3[0]
    cy = acc3[1]
    cz = acc3[2]
    sn = (cx * cx + cy * cy) + cz * cz
    newb_ref[...] = jnp.stack(
        [_rtne_bf16(cx), _rtne_bf16(cy), _rtne_bf16(cz), sn, cx, cy, cz,
         jnp.zeros_like(cx)], axis=1)


def _run_fps(xyz, interpret=False):
    return pl.pallas_call(
        _fps_kernel,
        out_shape=(
            jax.ShapeDtypeStruct((B, 3, NPOINT), jnp.float32),
            jax.ShapeDtypeStruct((B, 4, N), jnp.float32),
            jax.ShapeDtypeStruct((B, 8, NPOINT), jnp.float32),
        ),
        scratch_shapes=[pltpu.VMEM((B, N), jnp.float32)],
        interpret=interpret,
    )(xyz)


R2 = RADIUS ** 2
NCHUNK = N // L
PLANE_W = ROWS_PER_W * NSAMPLE


def _sc_body(xyzb_hbm, newb_hbm, payload_hbm, out_hbm,
             xyzb_v, newb_v, payload_v, rowbuf, plane_v):
    cid = lax.axis_index("c")
    sid = lax.axis_index("s")
    wid = sid * NC + cid
    b = wid // 2
    half = wid % 2

    pltpu.sync_copy(xyzb_hbm.at[b], xyzb_v)
    pltpu.sync_copy(newb_hbm.at[b, half], newb_v)
    pltpu.sync_copy(payload_hbm.at[b], payload_v)

    iota16 = lax.iota(jnp.int32, 16)
    KSTRIDE = 6 * ROWS_PER_W
    kdest0 = iota16 * KSTRIDE
    kdest1 = (iota16 + L) * KSTRIDE
    UB = 8
    RB = NSAMPLE + L

    def row_body(r, dummy):
        def bc(ch):
            return plsc.load_gather(newb_v, [jnp.full((L,), ch * ROWS_PER_W, jnp.int32) + r])

        sxb, syb, szb, snv = bc(0), bc(1), bc(2), bc(3)
        sx, sy, sz = bc(4), bc(5), bc(6)

        def cond(carry):
            c, cnt, cntv = carry
            return (c < NCHUNK) & (cnt < NSAMPLE)

        def body(carry):
            c, cnt, cntv = carry
            for u in range(UB):
                base = (c + u) * L
                px = xyzb_v[pl.ds(base, L)]
                py = xyzb_v[pl.ds(N + base, L)]
                pz = xyzb_v[pl.ds(2 * N + base, L)]
                tn = xyzb_v[pl.ds(3 * N + base, L)]
                dot = sxb * px + syb * py
                dot = dot + szb * pz
                dd = (jnp.float32(-2.0) * dot + snv) + tn
                mask = dd <= R2
                v = mask.astype(jnp.int32)
                for s in (1, 2, 4, 8):
                    sh = _vec_take(v, jnp.maximum(iota16 - s, 0))
                    v = v + jnp.where(iota16 >= s, sh, 0)
                dest = (cntv + v) - 1
                wmask = mask & (dest < RB)
                plsc.store_scatter(rowbuf, [dest], base + iota16, mask=wmask)
                cntv = cntv + plsc.all_reduce_population_count(mask)
            return c + UB, jnp.max(cntv), cntv

        _, _, cntv = lax.while_loop(
            cond, body,
            (jnp.int32(0), jnp.int32(0), jnp.zeros((L,), jnp.int32)))

        first = plsc.load_gather(rowbuf, [jnp.zeros((L,), jnp.int32)])
        v0 = jnp.where(iota16 < cntv, rowbuf[pl.ds(0, L)], first)
        v1 = jnp.where((iota16 + L) < cntv, rowbuf[pl.ds(L, L)], first)

        for c in range(6):
            sub = (sx, sy, sz, None, None, None)[c]
            cbase = c * ROWS_PER_W + r
            for vv, kd in ((v0, kdest0), (v1, kdest1)):
                val = plsc.load_gather(payload_v, [vv + (c * N)])
                if sub is not None:
                    val = val - sub
                plsc.store_scatter(plane_v, [kd + cbase], val)
        return dummy

    lax.fori_loop(0, ROWS_PER_W, row_body, jnp.int32(0))

    obase = wid * ROWS_PER_W
    for k in range(NSAMPLE):
        for c in range(6):
            pltpu.sync_copy(
                plane_v.at[pl.ds((k * 6 + c) * ROWS_PER_W, ROWS_PER_W)],
                out_hbm.at[k, c, pl.ds(obase, ROWS_PER_W)])


def _run_sc_ball_gather(xyzb_flat, newb_flat, payload_flat):
    mesh = plsc.VectorSubcoreMesh(core_axis_name="c", subcore_axis_name="s")
    k = pl.kernel(
        _sc_body,
        out_type=jax.ShapeDtypeStruct((NSAMPLE, 6, B * NPOINT), jnp.float32),
        mesh=mesh,
        compiler_params=pltpu.CompilerParams(needs_layout_passes=False),
        scratch_types=[
            pltpu.VMEM((4 * N,), jnp.float32),
            pltpu.VMEM((8 * ROWS_PER_W,), jnp.float32),
            pltpu.VMEM((6 * N,), jnp.float32),
            pltpu.VMEM((NSAMPLE + L,), jnp.int32),
            pltpu.VMEM((6 * PLANE_W,), jnp.float32),
        ],
    )
    return k(xyzb_flat, newb_flat, payload_flat)


R_TOTAL = B * NPOINT * NSAMPLE
BS = B * NPOINT
CT = 8192
NCT = BS // CT


def _bn_relu(y, stats, g, bt):
    m = stats[:, 0:1] / R_TOTAL
    v = stats[:, 1:2] / R_TOTAL - m * m
    return jnp.maximum(g * (y - m) * lax.rsqrt(v + EPS) + bt, 0.0)


def _mlp_chain(w_refs, stat_refs, x, nlayers):
    y = x
    for li in range(nlayers):
        if li > 0:
            g, bt = w_refs[4 * li - 2], w_refs[4 * li - 1]
            y = _bn_relu(y, stat_refs[li - 1][...], g[...], bt[...])
        W, b = w_refs[4 * li], w_refs[4 * li + 1]
        y = lax.dot_general(W[...], y, (((1,), (0,)), ((), ())),
                            preferred_element_type=jnp.float32) + b[...]
    return y


def _make_stats_kernel(nlayer, nw, ns):
    def kern(*refs):
        w_refs = refs[:nw]
        stat_refs = refs[nw:nw + ns]
        x_ref, out_ref, acc_ref = refs[nw + ns:]
        i = pl.program_id(0)
        y = _mlp_chain(w_refs, stat_refs, x_ref[0], nlayer + 1)

        @pl.when(i == 0)
        def _():
            acc_ref[...] = jnp.zeros_like(acc_ref)

        acc_ref[:, 0:1] += jnp.sum(y, axis=1, keepdims=True)
        acc_ref[:, 1:2] += jnp.sum(y * y, axis=1, keepdims=True)

        @pl.when(i == NSAMPLE - 1)
        def _():
            out_ref[...] = acc_ref[...]

    return kern


def _final_kernel(*refs):
    w_refs = refs[:12]
    stat_refs = refs[12:15]
    x_ref, out_ref, acc_ref = refs[15:]
    k = pl.program_id(1)
    y = _mlp_chain(w_refs, stat_refs, x_ref[0], 3)
    y = _bn_relu(y, stat_refs[2][...], w_refs[10][...], w_refs[11][...])

    @pl.when(k == 0)
    def _():
        acc_ref[...] = y

    @pl.when(k > 0)
    def _():
        acc_ref[...] = jnp.maximum(acc_ref[...], y)

    @pl.when(k == NSAMPLE - 1)
    def _():
        out_ref[...] = acc_ref[...]


def _full_spec(a):
    nd = a.ndim
    return pl.BlockSpec(a.shape, lambda i, _nd=nd: (0,) * _nd)


def _run_mlp(xplanes, params, interpret=False):
    w_args = []
    for (W, bias, g, bt) in params:
        w_args += [W, bias, g, bt]
    w_specs = [_full_spec(a) for a in w_args]

    stats = []
    for li in range(3):
        oc = params[li][0].shape[0]
        nw = 4 * li + 2
        st = pl.pallas_call(
            _make_stats_kernel(li, nw, li),
            grid=(NSAMPLE,),
            in_specs=w_specs[:nw]
            + [_full_spec(s) for s in stats]
            + [pl.BlockSpec((1, 6, BS), lambda i: (i, 0, 0))],
            out_specs=pl.BlockSpec((oc, 2), lambda i: (0, 0)),
            out_shape=jax.ShapeDtypeStruct((oc, 2), jnp.float32),
            scratch_shapes=[pltpu.VMEM((oc, 2), jnp.float32)],
            interpret=interpret,
        )(*w_args[:nw], *stats, xplanes)
        stats.append(st)

    out = pl.pallas_call(
        _final_kernel,
        grid=(NCT, NSAMPLE),
        in_specs=[pl.BlockSpec(s.block_shape, lambda i, k, _f=s.index_map: _f(i))
                  for s in w_specs]
        + [pl.BlockSpec(st.shape, lambda i, k, _nd=st.ndim: (0,) * _nd)
           for st in stats]
        + [pl.BlockSpec((1, 6, CT), lambda i, k: (k, 0, i))],
        out_specs=pl.BlockSpec((128, CT), lambda i, k: (0, i)),
        out_shape=jax.ShapeDtypeStruct((128, BS), jnp.float32),
        scratch_shapes=[pltpu.VMEM((128, CT), jnp.float32)],
        interpret=interpret,
    )(*w_args, *stats, xplanes)
    return out


def kernel(xyz, points, W0, b0, g0, beta0, W1, b1, g1, beta1, W2, b2, g2, beta2,
           interpret=False):
    new_xyz_t, xyzb, newb = _run_fps(xyz, interpret=interpret)
    xyzb_flat = xyzb.reshape(B, 4 * N)
    newb_flat = newb.reshape(B, 8, 2, NPOINT // 2).transpose(0, 2, 1, 3).reshape(
        B, 2, 8 * (NPOINT // 2))
    payload_flat = jnp.concatenate([xyz, points], axis=1).reshape(B, 6 * N)
    xplanes = _run_sc_ball_gather(xyzb_flat, newb_flat, payload_flat)
    params = [
        (W0, b0[:, None], g0[:, None], beta0[:, None]),
        (W1, b1[:, None], g1[:, None], beta1[:, None]),
        (W2, b2[:, None], g2[:, None], beta2[:, None]),
    ]
    pooled = _run_mlp(xplanes, params, interpret=interpret)
    new_points = pooled.reshape(128, B, NPOINT).transpose(1, 0, 2)
    return new_xyz_t, new_points


# speedup vs baseline: 2.5755x; 2.5755x over previous
import functools

import jax
import jax.numpy as jnp
from jax import lax
from jax.experimental import pallas as pl
from jax.experimental.pallas import tpu as pltpu
from jax.experimental.pallas import tpu_sc as plsc

NPOINT = 512
RADIUS = 0.2
NSAMPLE = 32
B = 16
N = 4096
EPS = 1e-5

NC, NS, L = 2, 16, 16
NW = NC * NS
ROWS_PER_W = B * NPOINT // NW


def _vec_take(v, idx):
    return lax.gather(
        v, idx[:, None],
        lax.GatherDimensionNumbers(offset_dims=(), collapsed_slice_dims=(0,),
                                   start_index_map=(0,)),
        slice_sizes=(1,),
        mode=lax.GatherScatterMode.PROMISE_IN_BOUNDS)


def _rtne_bf16(v):
    u = lax.bitcast_convert_type(v, jnp.uint32)
    u2 = u + jnp.uint32(0x7FFF) + ((u >> 16) & jnp.uint32(1))
    return lax.bitcast_convert_type(u2 & jnp.uint32(0xFFFF0000), jnp.float32)


def _fps_kernel(xyz_ref, out_ref, xyzb_ref, newb_ref, dist_ref):
    x = xyz_ref[:, 0, :]
    y = xyz_ref[:, 1, :]
    z = xyz_ref[:, 2, :]
    p48 = jnp.concatenate([x, y, z], axis=0)
    iota = lax.broadcasted_iota(jnp.int32, (B, N), 1)
    siota3 = lax.broadcasted_iota(jnp.int32, (3, B, NPOINT), 2)
    big = jnp.int32(N)

    m0 = jnp.max(x, axis=1, keepdims=True)
    far0 = jnp.min(jnp.where(x == m0, iota, big), axis=1, keepdims=True)
    dist_ref[...] = jnp.full((B, N), 1e10, dtype=jnp.float32)
    acc0 = jnp.zeros((3, B, NPOINT), jnp.float32)

    def body(i, carry):
        far, acc3 = carry
        onehot = iota == far
        oh3 = jnp.concatenate([onehot, onehot, onehot], axis=0)
        ext = jnp.sum(jnp.where(oh3, p48, 0.0), axis=1, keepdims=True)
        acc3 = jnp.where(siota3 == i, ext.reshape(3, B, 1), acc3)
        diff = p48 - ext
        sq = diff * diff
        d = (sq[0:B] + sq[B:2 * B]) + sq[2 * B:3 * B]
        dist = dist_ref[...]
        dist = jnp.where(d < dist, d, dist)
        dist_ref[...] = dist
        m = jnp.max(dist, axis=1, keepdims=True)
        far = jnp.min(jnp.where(dist == m, iota, big), axis=1, keepdims=True)
        return far, acc3

    _, acc3 = lax.fori_loop(0, NPOINT, body, (far0, acc0))
    out_ref[...] = jnp.stack([acc3[0], acc3[1], acc3[2]], axis=1)

    tn = (x * x + y * y) + z * z
    xyzb_ref[...] = jnp.stack([_rtne_bf16(x), _rtne_bf16(y), _rtne_bf16(z), tn],
                              axis=1)
    cx = acc3[0]
    cy = acc3[1]
    cz = acc3[2]
    sn = (cx * cx + cy * cy) + cz * cz
    newb_ref[...] = jnp.stack(
        [_rtne_bf16(cx), _rtne_bf16(cy), _rtne_bf16(cz), sn, cx, cy, cz,
         jnp.zeros_like(cx)], axis=1)


def _run_fps(xyz, interpret=False):
    return pl.pallas_call(
        _fps_kernel,
        out_shape=(
            jax.ShapeDtypeStruct((B, 3, NPOINT), jnp.float32),
            jax.ShapeDtypeStruct((B, 4, N), jnp.float32),
            jax.ShapeDtypeStruct((B, 8, NPOINT), jnp.float32),
        ),
        scratch_shapes=[pltpu.VMEM((B, N), jnp.float32)],
        interpret=interpret,
    )(xyz)


R2 = RADIUS ** 2
NCHUNK = N // L
PLANE_W = ROWS_PER_W * NSAMPLE


def _sc_body(xyzb_hbm, newb_hbm, payload_hbm, out_hbm,
             xyzb_v, newb_v, payload_v, rowbuf, plane_v):
    cid = lax.axis_index("c")
    sid = lax.axis_index("s")
    wid = sid * NC + cid
    b = wid // 2
    half = wid % 2

    pltpu.sync_copy(xyzb_hbm.at[b], xyzb_v)
    pltpu.sync_copy(newb_hbm.at[b, half], newb_v)
    pltpu.sync_copy(payload_hbm.at[b], payload_v)

    iota16 = lax.iota(jnp.int32, 16)
    KSTRIDE = 6 * ROWS_PER_W
    kdest0 = iota16 * KSTRIDE
    kdest1 = (iota16 + L) * KSTRIDE
    UB = 8
    RB = NSAMPLE + L

    def row_body(r, dummy):
        def bc(ch):
            return plsc.load_gather(newb_v, [jnp.full((L,), ch * ROWS_PER_W, jnp.int32) + r])

        sxb, syb, szb, snv = bc(0), bc(1), bc(2), bc(3)
        sx, sy, sz = bc(4), bc(5), bc(6)

        def cond(carry):
            c, cnt, cntv = carry
            return (c < NCHUNK) & (cnt < NSAMPLE)

        def body(carry):
            c, cnt, cntv = carry
            for u in range(UB):
                base = (c + u) * L
                px = xyzb_v[pl.ds(base, L)]
                py = xyzb_v[pl.ds(N + base, L)]
                pz = xyzb_v[pl.ds(2 * N + base, L)]
                tn = xyzb_v[pl.ds(3 * N + base, L)]
                dot = sxb * px + syb * py
                dot = dot + szb * pz
                dd = (jnp.float32(-2.0) * dot + snv) + tn
                mask = dd <= R2
                v = mask.astype(jnp.int32)
                for s in (1, 2, 4, 8):
                    sh = _vec_take(v, jnp.maximum(iota16 - s, 0))
                    v = v + jnp.where(iota16 >= s, sh, 0)
                dest = (cntv + v) - 1
                wmask = mask & (dest < RB)
                plsc.store_scatter(rowbuf, [dest], base + iota16, mask=wmask)
                cntv = cntv + plsc.all_reduce_population_count(mask)
            return c + UB, jnp.max(cntv), cntv

        _, _, cntv = lax.while_loop(
            cond, body,
            (jnp.int32(0), jnp.int32(0), jnp.zeros((L,), jnp.int32)))

        first = plsc.load_gather(rowbuf, [jnp.zeros((L,), jnp.int32)])
        v0 = jnp.where(iota16 < cntv, rowbuf[pl.ds(0, L)], first)
        v1 = jnp.where((iota16 + L) < cntv, rowbuf[pl.ds(L, L)], first)

        for c in range(6):
            sub = (sx, sy, sz, None, None, None)[c]
            cbase = c * ROWS_PER_W + r
            for vv, kd in ((v0, kdest0), (v1, kdest1)):
                val = plsc.load_gather(payload_v, [vv + (c * N)])
                if sub is not None:
                    val = val - sub
                plsc.store_scatter(plane_v, [kd + cbase], val)
        return dummy

    lax.fori_loop(0, ROWS_PER_W, row_body, jnp.int32(0))

    obase = wid * ROWS_PER_W
    for k in range(NSAMPLE):
        for c in range(6):
            pltpu.sync_copy(
                plane_v.at[pl.ds((k * 6 + c) * ROWS_PER_W, ROWS_PER_W)],
                out_hbm.at[k, c, pl.ds(obase, ROWS_PER_W)])


def _run_sc_ball_gather(xyzb_flat, newb_flat, payload_flat):
    mesh = plsc.VectorSubcoreMesh(core_axis_name="c", subcore_axis_name="s")
    k = pl.kernel(
        _sc_body,
        out_type=jax.ShapeDtypeStruct((NSAMPLE, 6, B * NPOINT), jnp.float32),
        mesh=mesh,
        compiler_params=pltpu.CompilerParams(needs_layout_passes=False),
        scratch_types=[
            pltpu.VMEM((4 * N,), jnp.float32),
            pltpu.VMEM((8 * ROWS_PER_W,), jnp.float32),
            pltpu.VMEM((6 * N,), jnp.float32),
            pltpu.VMEM((NSAMPLE + L,), jnp.int32),
            pltpu.VMEM((6 * PLANE_W,), jnp.float32),
        ],
    )
    return k(xyzb_flat, newb_flat, payload_flat)


R_TOTAL = B * NPOINT * NSAMPLE
BS = B * NPOINT
CT = 8192
NCT = BS // CT


def _bn_relu(y, stats, g, bt):
    m = stats[:, 0:1] / R_TOTAL
    v = stats[:, 1:2] / R_TOTAL - m * m
    return jnp.maximum(g * (y - m) * lax.rsqrt(v + EPS) + bt, 0.0)


def _mlp_chain(w_refs, stat_refs, x, nlayers):
    y = x
    for li in range(nlayers):
        if li > 0:
            g, bt = w_refs[4 * li - 2], w_refs[4 * li - 1]
            y = _bn_relu(y, stat_refs[li - 1][...], g[...], bt[...])
        W, b = w_refs[4 * li], w_refs[4 * li + 1]
        y = lax.dot_general(W[...], y, (((1,), (0,)), ((), ())),
                            preferred_element_type=jnp.float32) + b[...]
    return y


def _make_stats_kernel(nlayer, nw, ns):
    def kern(*refs):
        w_refs = refs[:nw]
        stat_refs = refs[nw:nw + ns]
        x_ref, out_ref, acc_ref = refs[nw + ns:]
        i = pl.program_id(0)
        y = _mlp_chain(w_refs, stat_refs, x_ref[0], nlayer + 1)

        @pl.when(i == 0)
        def _():
            acc_ref[...] = jnp.zeros_like(acc_ref)

        acc_ref[:, 0:1] += jnp.sum(y, axis=1, keepdims=True)
        acc_ref[:, 1:2] += jnp.sum(y * y, axis=1, keepdims=True)

        @pl.when(i == NSAMPLE - 1)
        def _():
            out_ref[...] = acc_ref[...]

    return kern


def _final_kernel(*refs):
    w_refs = refs[:12]
    stat_refs = refs[12:15]
    x_ref, out_ref, acc_ref = refs[15:]
    k = pl.program_id(1)
    y = _mlp_chain(w_refs, stat_refs, x_ref[0], 3)
    y = _bn_relu(y, stat_refs[2][...], w_refs[10][...], w_refs[11][...])

    @pl.when(k == 0)
    def _():
        acc_ref[...] = y

    @pl.when(k > 0)
    def _():
        acc_ref[...] = jnp.maximum(acc_ref[...], y)

    @pl.when(k == NSAMPLE - 1)
    def _():
        out_ref[...] = acc_ref[...]


def _full_spec(a):
    nd = a.ndim
    return pl.BlockSpec(a.shape, lambda i, _nd=nd: (0,) * _nd)


def _run_mlp(xplanes, params, interpret=False):
    w_args = []
    for (W, bias, g, bt) in params:
        w_args += [W, bias, g, bt]
    w_specs = [_full_spec(a) for a in w_args]

    stats = []
    for li in range(3):
        oc = params[li][0].shape[0]
        nw = 4 * li + 2
        st = pl.pallas_call(
            _make_stats_kernel(li, nw, li),
            grid=(NSAMPLE,),
            in_specs=w_specs[:nw]
            + [_full_spec(s) for s in stats]
            + [pl.BlockSpec((1, 6, BS), lambda i: (i, 0, 0))],
            out_specs=pl.BlockSpec((oc, 2), lambda i: (0, 0)),
            out_shape=jax.ShapeDtypeStruct((oc, 2), jnp.float32),
            scratch_shapes=[pltpu.VMEM((oc, 2), jnp.float32)],
            interpret=interpret,
        )(*w_args[:nw], *stats, xplanes)
        stats.append(st)

    out = pl.pallas_call(
        _final_kernel,
        grid=(NCT, NSAMPLE),
        in_specs=[pl.BlockSpec(s.block_shape, lambda i, k, _f=s.index_map: _f(i))
                  for s in w_specs]
        + [pl.BlockSpec(st.shape, lambda i, k, _nd=st.ndim: (0,) * _nd)
           for st in stats]
        + [pl.BlockSpec((1, 6, CT), lambda i, k: (k, 0, i))],
        out_specs=pl.BlockSpec((128, CT), lambda i, k: (0, i)),
        out_shape=jax.ShapeDtypeStruct((128, BS), jnp.float32),
        scratch_shapes=[pltpu.VMEM((128, CT), jnp.float32)],
        interpret=interpret,
    )(*w_args, *stats, xplanes)
    return out


def kernel(xyz, points, W0, b0, g0, beta0, W1, b1, g1, beta1, W2, b2, g2, beta2,
           interpret=False):
    new_xyz_t, xyzb, newb = _run_fps(xyz, interpret=interpret)
    xyzb_flat = xyzb.reshape(B, 4 * N)
    newb_flat = newb.reshape(B, 8, 2, NPOINT // 2).transpose(0, 2, 1, 3).reshape(
        B, 2, 8 * (NPOINT // 2))
    payload_flat = jnp.concatenate([xyz, points], axis=1).reshape(B, 6 * N)
    xplanes = _run_sc_ball_gather(xyzb_flat, newb_flat, payload_flat)
    params = [
        (W0, b0[:, None], g0[:, None], beta0[:, None]),
        (W1, b1[:, None], g1[:, None], beta1[:, None]),
        (W2, b2[:, None], g2[:, None], beta2[:, None]),
    ]
    pooled = _run_mlp(xplanes, params, interpret=interpret)
    new_points = pooled.reshape(128, B, NPOINT).transpose(1, 0, 2)
    return new_xyz_t, new_points


# speedup vs baseline: 3.4692x; 1.3470x over previous
import functools

import jax
import jax.numpy as jnp
from jax import lax
from jax.experimental import pallas as pl
from jax.experimental.pallas import tpu as pltpu
from jax.experimental.pallas import tpu_sc as plsc

NPOINT = 512
RADIUS = 0.2
NSAMPLE = 32
B = 16
N = 4096
EPS = 1e-5

NC, NS, L = 2, 16, 16
NW = NC * NS
ROWS_PER_W = B * NPOINT // NW


def _vec_take(v, idx):
    return lax.gather(
        v, idx[:, None],
        lax.GatherDimensionNumbers(offset_dims=(), collapsed_slice_dims=(0,),
                                   start_index_map=(0,)),
        slice_sizes=(1,),
        mode=lax.GatherScatterMode.PROMISE_IN_BOUNDS)


def _rtne_bf16(v):
    u = lax.bitcast_convert_type(v, jnp.uint32)
    u2 = u + jnp.uint32(0x7FFF) + ((u >> 16) & jnp.uint32(1))
    return lax.bitcast_convert_type(u2 & jnp.uint32(0xFFFF0000), jnp.float32)


def _fps_kernel(xyz_ref, out_ref, xyzb_ref, newb_ref, dist_ref):
    x = xyz_ref[:, 0, :]
    y = xyz_ref[:, 1, :]
    z = xyz_ref[:, 2, :]
    p48 = jnp.concatenate([x, y, z], axis=0)
    iota = lax.broadcasted_iota(jnp.int32, (B, N), 1)
    siota3 = lax.broadcasted_iota(jnp.int32, (3, B, NPOINT), 2)
    big = jnp.int32(N)

    m0 = jnp.max(x, axis=1, keepdims=True)
    far0 = jnp.min(jnp.where(x == m0, iota, big), axis=1, keepdims=True)
    dist_ref[...] = jnp.full((B, N), 1e10, dtype=jnp.float32)
    acc0 = jnp.zeros((3, B, NPOINT), jnp.float32)

    def body(i, carry):
        far, acc3 = carry
        onehot = iota == far
        oh3 = jnp.concatenate([onehot, onehot, onehot], axis=0)
        ext = jnp.sum(jnp.where(oh3, p48, 0.0), axis=1, keepdims=True)
        acc3 = jnp.where(siota3 == i, ext.reshape(3, B, 1), acc3)
        diff = p48 - ext
        sq = diff * diff
        d = (sq[0:B] + sq[B:2 * B]) + sq[2 * B:3 * B]
        dist = dist_ref[...]
        dist = jnp.where(d < dist, d, dist)
        dist_ref[...] = dist
        m = jnp.max(dist, axis=1, keepdims=True)
        far = jnp.min(jnp.where(dist == m, iota, big), axis=1, keepdims=True)
        return far, acc3

    _, acc3 = lax.fori_loop(0, NPOINT, body, (far0, acc0))
    out_ref[...] = jnp.stack([acc3[0], acc3[1], acc3[2]], axis=1)

    tn = (x * x + y * y) + z * z
    xyzb_ref[...] = jnp.stack([_rtne_bf16(x), _rtne_bf16(y), _rtne_bf16(z), tn],
                              axis=1)
    cx = acc3[0]
    cy = acc3[1]
    cz = acc3[2]
    sn = (cx * cx + cy * cy) + cz * cz
    newb_ref[...] = jnp.stack(
        [_rtne_bf16(cx), _rtne_bf16(cy), _rtne_bf16(cz), sn, cx, cy, cz,
         jnp.zeros_like(cx)], axis=1)


def _run_fps(xyz, interpret=False):
    return pl.pallas_call(
        _fps_kernel,
        out_shape=(
            jax.ShapeDtypeStruct((B, 3, NPOINT), jnp.float32),
            jax.ShapeDtypeStruct((B, 4, N), jnp.float32),
            jax.ShapeDtypeStruct((B, 8, NPOINT), jnp.float32),
        ),
        scratch_shapes=[pltpu.VMEM((B, N), jnp.float32)],
        interpret=interpret,
    )(xyz)


R2 = RADIUS ** 2
NCHUNK = N // L
PLANE_W = ROWS_PER_W * NSAMPLE


def _sc_body(xyzb_hbm, newb_hbm, payload_hbm, out_hbm,
             xyzb_v, newb_v, payload_v, rowbuf, plane_v):
    cid = lax.axis_index("c")
    sid = lax.axis_index("s")
    wid = sid * NC + cid
    b = wid // 2
    half = wid % 2

    pltpu.sync_copy(xyzb_hbm.at[b], xyzb_v)
    pltpu.sync_copy(newb_hbm.at[b, half], newb_v)
    pltpu.sync_copy(payload_hbm.at[b], payload_v)

    iota16 = lax.iota(jnp.int32, 16)
    KSTRIDE = 6 * ROWS_PER_W
    kdest0 = iota16 * KSTRIDE
    kdest1 = (iota16 + L) * KSTRIDE
    UB = 8
    RB = NSAMPLE + L

    HALF_R = ROWS_PER_W // 2

    def pair_body(r, dummy):
        rows = (r, r + HALF_R)

        def bc(ch, rr):
            return plsc.load_gather(
                newb_v, [jnp.full((L,), ch * ROWS_PER_W, jnp.int32) + rr])

        sv = [tuple(bc(ch, rr) for ch in range(7)) for rr in rows]

        def cond(carry):
            c, cnt0, cnt1 = carry[:3]
            return (c < NCHUNK) & ((cnt0 < NSAMPLE) | (cnt1 < NSAMPLE))

        def body(carry):
            c, _, _, cntv0, cntv1 = carry
            cntvs = [cntv0, cntv1]
            for u in range(UB):
                base = (c + u) * L
                px = xyzb_v[pl.ds(base, L)]
                py = xyzb_v[pl.ds(N + base, L)]
                pz = xyzb_v[pl.ds(2 * N + base, L)]
                tn = xyzb_v[pl.ds(3 * N + base, L)]
                idxv = base + iota16
                for t in range(2):
                    sxb, syb, szb, snv = sv[t][:4]
                    dot = sxb * px + syb * py
                    dot = dot + szb * pz
                    dd = (jnp.float32(-2.0) * dot + snv) + tn
                    mask = dd <= R2
                    incl = plsc.cumsum(mask.astype(jnp.int32))
                    dest = (cntvs[t] + incl) - 1
                    wmask = mask & (dest < RB)
                    plsc.store_scatter(rowbuf, [dest + t * RB], idxv,
                                       mask=wmask)
                    cntvs[t] = cntvs[t] + plsc.all_reduce_population_count(mask)
            return (c + UB, jnp.max(cntvs[0]), jnp.max(cntvs[1]),
                    cntvs[0], cntvs[1])

        zero = jnp.zeros((L,), jnp.int32)
        _, _, _, cntv0, cntv1 = lax.while_loop(
            cond, body, (jnp.int32(0), jnp.int32(0), jnp.int32(0), zero, zero))

        for t, rr, cntv in ((0, rows[0], cntv0), (1, rows[1], cntv1)):
            sx, sy, sz = sv[t][4:7]
            off = t * RB
            first = plsc.load_gather(rowbuf, [jnp.full((L,), off, jnp.int32)])
            v0 = jnp.where(iota16 < cntv, rowbuf[pl.ds(off, L)], first)
            v1 = jnp.where((iota16 + L) < cntv, rowbuf[pl.ds(off + L, L)],
                           first)

            for c in range(6):
                sub = (sx, sy, sz, None, None, None)[c]
                cbase = c * ROWS_PER_W + rr
                for vv, kd in ((v0, kdest0), (v1, kdest1)):
                    val = plsc.load_gather(payload_v, [vv + (c * N)])
                    if sub is not None:
                        val = val - sub
                    plsc.store_scatter(plane_v, [kd + cbase], val)
        return dummy

    lax.fori_loop(0, HALF_R, pair_body, jnp.int32(0))

    obase = wid * ROWS_PER_W
    for k in range(NSAMPLE):
        for c in range(6):
            pltpu.sync_copy(
                plane_v.at[pl.ds((k * 6 + c) * ROWS_PER_W, ROWS_PER_W)],
                out_hbm.at[k, c, pl.ds(obase, ROWS_PER_W)])


def _run_sc_ball_gather(xyzb_flat, newb_flat, payload_flat):
    mesh = plsc.VectorSubcoreMesh(core_axis_name="c", subcore_axis_name="s")
    k = pl.kernel(
        _sc_body,
        out_type=jax.ShapeDtypeStruct((NSAMPLE, 6, B * NPOINT), jnp.float32),
        mesh=mesh,
        compiler_params=pltpu.CompilerParams(needs_layout_passes=False),
        scratch_types=[
            pltpu.VMEM((4 * N,), jnp.float32),
            pltpu.VMEM((8 * ROWS_PER_W,), jnp.float32),
            pltpu.VMEM((6 * N,), jnp.float32),
            pltpu.VMEM((2 * (NSAMPLE + L),), jnp.int32),
            pltpu.VMEM((6 * PLANE_W,), jnp.float32),
        ],
    )
    return k(xyzb_flat, newb_flat, payload_flat)


R_TOTAL = B * NPOINT * NSAMPLE
BS = B * NPOINT
CT = 8192
NCT = BS // CT


def _bn_relu(y, stats, g, bt):
    m = stats[:, 0:1] / R_TOTAL
    v = stats[:, 1:2] / R_TOTAL - m * m
    return jnp.maximum(g * (y - m) * lax.rsqrt(v + EPS) + bt, 0.0)


def _mlp_chain(w_refs, stat_refs, x, nlayers):
    y = x
    for li in range(nlayers):
        if li > 0:
            g, bt = w_refs[4 * li - 2], w_refs[4 * li - 1]
            y = _bn_relu(y, stat_refs[li - 1][...], g[...], bt[...])
        W, b = w_refs[4 * li], w_refs[4 * li + 1]
        y = lax.dot_general(W[...], y, (((1,), (0,)), ((), ())),
                            preferred_element_type=jnp.float32) + b[...]
    return y


def _make_stats_kernel(nlayer, nw, ns):
    def kern(*refs):
        w_refs = refs[:nw]
        stat_refs = refs[nw:nw + ns]
        x_ref, out_ref, acc_ref = refs[nw + ns:]
        i = pl.program_id(0)
        y = _mlp_chain(w_refs, stat_refs, x_ref[0], nlayer + 1)

        @pl.when(i == 0)
        def _():
            acc_ref[...] = jnp.zeros_like(acc_ref)

        acc_ref[:, 0:1] += jnp.sum(y, axis=1, keepdims=True)
        acc_ref[:, 1:2] += jnp.sum(y * y, axis=1, keepdims=True)

        @pl.when(i == NSAMPLE - 1)
        def _():
            out_ref[...] = acc_ref[...]

    return kern


def _final_kernel(*refs):
    w_refs = refs[:12]
    stat_refs = refs[12:15]
    x_ref, out_ref, acc_ref = refs[15:]
    k = pl.program_id(1)
    y = _mlp_chain(w_refs, stat_refs, x_ref[0], 3)
    y = _bn_relu(y, stat_refs[2][...], w_refs[10][...], w_refs[11][...])

    @pl.when(k == 0)
    def _():
        acc_ref[...] = y

    @pl.when(k > 0)
    def _():
        acc_ref[...] = jnp.maximum(acc_ref[...], y)

    @pl.when(k == NSAMPLE - 1)
    def _():
        out_ref[...] = acc_ref[...]


def _full_spec(a):
    nd = a.ndim
    return pl.BlockSpec(a.shape, lambda i, _nd=nd: (0,) * _nd)


def _run_mlp(xplanes, params, interpret=False):
    w_args = []
    for (W, bias, g, bt) in params:
        w_args += [W, bias, g, bt]
    w_specs = [_full_spec(a) for a in w_args]

    stats = []
    for li in range(3):
        oc = params[li][0].shape[0]
        nw = 4 * li + 2
        st = pl.pallas_call(
            _make_stats_kernel(li, nw, li),
            grid=(NSAMPLE,),
            in_specs=w_specs[:nw]
            + [_full_spec(s) for s in stats]
            + [pl.BlockSpec((1, 6, BS), lambda i: (i, 0, 0))],
            out_specs=pl.BlockSpec((oc, 2), lambda i: (0, 0)),
            out_shape=jax.ShapeDtypeStruct((oc, 2), jnp.float32),
            scratch_shapes=[pltpu.VMEM((oc, 2), jnp.float32)],
            interpret=interpret,
        )(*w_args[:nw], *stats, xplanes)
        stats.append(st)

    out = pl.pallas_call(
        _final_kernel,
        grid=(NCT, NSAMPLE),
        in_specs=[pl.BlockSpec(s.block_shape, lambda i, k, _f=s.index_map: _f(i))
                  for s in w_specs]
        + [pl.BlockSpec(st.shape, lambda i, k, _nd=st.ndim: (0,) * _nd)
           for st in stats]
        + [pl.BlockSpec((1, 6, CT), lambda i, k: (k, 0, i))],
        out_specs=pl.BlockSpec((128, CT), lambda i, k: (0, i)),
        out_shape=jax.ShapeDtypeStruct((128, BS), jnp.float32),
        scratch_shapes=[pltpu.VMEM((128, CT), jnp.float32)],
        interpret=interpret,
    )(*w_args, *stats, xplanes)
    return out


def kernel(xyz, points, W0, b0, g0, beta0, W1, b1, g1, beta1, W2, b2, g2, beta2,
           interpret=False):
    new_xyz_t, xyzb, newb = _run_fps(xyz, interpret=interpret)
    xyzb_flat = xyzb.reshape(B, 4 * N)
    newb_flat = newb.reshape(B, 8, 2, NPOINT // 2).transpose(0, 2, 1, 3).reshape(
        B, 2, 8 * (NPOINT // 2))
    payload_flat = jnp.concatenate([xyz, points], axis=1).reshape(B, 6 * N)
    xplanes = _run_sc_ball_gather(xyzb_flat, newb_flat, payload_flat)
    params = [
        (W0, b0[:, None], g0[:, None], beta0[:, None]),
        (W1, b1[:, None], g1[:, None], beta1[:, None]),
        (W2, b2[:, None], g2[:, None], beta2[:, None]),
    ]
    pooled = _run_mlp(xplanes, params, interpret=interpret)
    new_points = pooled.reshape(128, B, NPOINT).transpose(1, 0, 2)
    return new_xyz_t, new_points


# speedup vs baseline: 3.9698x; 1.1443x over previous
import functools

import jax
import jax.numpy as jnp
from jax import lax
from jax.experimental import pallas as pl
from jax.experimental.pallas import tpu as pltpu
from jax.experimental.pallas import tpu_sc as plsc

NPOINT = 512
RADIUS = 0.2
NSAMPLE = 32
B = 16
N = 4096
EPS = 1e-5

NC, NS, L = 2, 16, 16
NW = NC * NS
ROWS_PER_W = B * NPOINT // NW


def _vec_take(v, idx):
    return lax.gather(
        v, idx[:, None],
        lax.GatherDimensionNumbers(offset_dims=(), collapsed_slice_dims=(0,),
                                   start_index_map=(0,)),
        slice_sizes=(1,),
        mode=lax.GatherScatterMode.PROMISE_IN_BOUNDS)


def _rtne_bf16(v):
    u = lax.bitcast_convert_type(v, jnp.uint32)
    u2 = u + jnp.uint32(0x7FFF) + ((u >> 16) & jnp.uint32(1))
    return lax.bitcast_convert_type(u2 & jnp.uint32(0xFFFF0000), jnp.float32)


def _fps_kernel(xyz_ref, out_ref, xyzb_ref, newb_ref, dist_ref):
    x = xyz_ref[:, 0, :]
    y = xyz_ref[:, 1, :]
    z = xyz_ref[:, 2, :]
    p48 = jnp.concatenate([x, y, z], axis=0)
    iota = lax.broadcasted_iota(jnp.int32, (B, N), 1)
    siota3 = lax.broadcasted_iota(jnp.int32, (3, B, NPOINT), 2)
    big = jnp.int32(N)

    m0 = jnp.max(x, axis=1, keepdims=True)
    far0 = jnp.min(jnp.where(x == m0, iota, big), axis=1, keepdims=True)
    dist_ref[...] = jnp.full((B, N), 1e10, dtype=jnp.float32)
    acc0 = jnp.zeros((3, B, NPOINT), jnp.float32)

    def body(i, carry):
        far, acc3 = carry
        onehot = iota == far
        oh3 = jnp.concatenate([onehot, onehot, onehot], axis=0)
        ext = jnp.sum(jnp.where(oh3, p48, 0.0), axis=1, keepdims=True)
        acc3 = jnp.where(siota3 == i, ext.reshape(3, B, 1), acc3)
        diff = p48 - ext
        sq = diff * diff
        d = (sq[0:B] + sq[B:2 * B]) + sq[2 * B:3 * B]
        dist = dist_ref[...]
        dist = jnp.where(d < dist, d, dist)
        dist_ref[...] = dist
        m = jnp.max(dist, axis=1, keepdims=True)
        far = jnp.min(jnp.where(dist == m, iota, big), axis=1, keepdims=True)
        return far, acc3

    _, acc3 = lax.fori_loop(0, NPOINT, body, (far0, acc0))
    out_ref[...] = jnp.stack([acc3[0], acc3[1], acc3[2]], axis=1)

    tn = (x * x + y * y) + z * z
    xyzb_ref[...] = jnp.stack([_rtne_bf16(x), _rtne_bf16(y), _rtne_bf16(z), tn],
                              axis=1)
    cx = acc3[0]
    cy = acc3[1]
    cz = acc3[2]
    sn = (cx * cx + cy * cy) + cz * cz
    newb_ref[...] = jnp.stack(
        [_rtne_bf16(cx), _rtne_bf16(cy), _rtne_bf16(cz), sn, cx, cy, cz,
         jnp.zeros_like(cx)], axis=1)


def _run_fps(xyz, interpret=False):
    return pl.pallas_call(
        _fps_kernel,
        out_shape=(
            jax.ShapeDtypeStruct((B, 3, NPOINT), jnp.float32),
            jax.ShapeDtypeStruct((B, 4, N), jnp.float32),
            jax.ShapeDtypeStruct((B, 8, NPOINT), jnp.float32),
        ),
        scratch_shapes=[pltpu.VMEM((B, N), jnp.float32)],
        interpret=interpret,
    )(xyz)


R2 = RADIUS ** 2
NCHUNK = N // L
PLANE_W = ROWS_PER_W * NSAMPLE


def _sc_body(xyzb_hbm, newb_hbm, payload_hbm, out_hbm,
             xyzb_v, newb_v, payload_v, rowbuf, plane_v):
    cid = lax.axis_index("c")
    sid = lax.axis_index("s")
    wid = sid * NC + cid
    b = wid // 2
    half = wid % 2

    pltpu.sync_copy(xyzb_hbm.at[b], xyzb_v)
    pltpu.sync_copy(newb_hbm.at[b, half], newb_v)
    pltpu.sync_copy(payload_hbm.at[b], payload_v)

    iota16 = lax.iota(jnp.int32, 16)
    KSTRIDE = 6 * ROWS_PER_W
    kdest0 = iota16 * KSTRIDE
    kdest1 = (iota16 + L) * KSTRIDE
    UB = 8
    RB = NSAMPLE + L

    NLANES_R = 4
    HALF_R = ROWS_PER_W // NLANES_R

    def pair_body(r, dummy):
        rows = tuple(r + t * HALF_R for t in range(NLANES_R))

        def bc(ch, rr):
            return plsc.load_gather(
                newb_v, [jnp.full((L,), ch * ROWS_PER_W, jnp.int32) + rr])

        sv = [tuple(bc(ch, rr) for ch in range(7)) for rr in rows]

        def cond(carry):
            c = carry[0]
            cnts = carry[1:1 + NLANES_R]
            active = cnts[0] < NSAMPLE
            for t in range(1, NLANES_R):
                active = active | (cnts[t] < NSAMPLE)
            return (c < NCHUNK) & active

        def body(carry):
            c = carry[0]
            cntvs = list(carry[1 + NLANES_R:])
            for u in range(UB):
                base = (c + u) * L
                px = xyzb_v[pl.ds(base, L)]
                py = xyzb_v[pl.ds(N + base, L)]
                pz = xyzb_v[pl.ds(2 * N + base, L)]
                tn = xyzb_v[pl.ds(3 * N + base, L)]
                idxv = base + iota16
                for t in range(NLANES_R):
                    sxb, syb, szb, snv = sv[t][:4]
                    dot = sxb * px + syb * py
                    dot = dot + szb * pz
                    dd = (jnp.float32(-2.0) * dot + snv) + tn
                    mask = dd <= R2
                    incl = plsc.cumsum(mask.astype(jnp.int32))
                    dest = (cntvs[t] + incl) - 1
                    wmask = mask & (dest < RB)
                    plsc.store_scatter(rowbuf, [dest + t * RB], idxv,
                                       mask=wmask)
                    cntvs[t] = cntvs[t] + plsc.all_reduce_population_count(mask)
            return ((c + UB,) + tuple(jnp.max(cv) for cv in cntvs)
                    + tuple(cntvs))

        zero = jnp.zeros((L,), jnp.int32)
        fin = lax.while_loop(
            cond, body,
            (jnp.int32(0),) + (jnp.int32(0),) * NLANES_R + (zero,) * NLANES_R)
        cntvs_fin = fin[1 + NLANES_R:]

        for t, rr, cntv in [(t, rows[t], cntvs_fin[t]) for t in range(NLANES_R)]:
            sx, sy, sz = sv[t][4:7]
            off = t * RB
            first = plsc.load_gather(rowbuf, [jnp.full((L,), off, jnp.int32)])
            v0 = jnp.where(iota16 < cntv, rowbuf[pl.ds(off, L)], first)
            v1 = jnp.where((iota16 + L) < cntv, rowbuf[pl.ds(off + L, L)],
                           first)

            for c in range(6):
                sub = (sx, sy, sz, None, None, None)[c]
                cbase = c * ROWS_PER_W + rr
                for vv, kd in ((v0, kdest0), (v1, kdest1)):
                    val = plsc.load_gather(payload_v, [vv + (c * N)])
                    if sub is not None:
                        val = val - sub
                    plsc.store_scatter(plane_v, [kd + cbase], val)
        return dummy

    lax.fori_loop(0, HALF_R, pair_body, jnp.int32(0))

    obase = wid * ROWS_PER_W
    for k in range(NSAMPLE):
        for c in range(6):
            pltpu.sync_copy(
                plane_v.at[pl.ds((k * 6 + c) * ROWS_PER_W, ROWS_PER_W)],
                out_hbm.at[k, c, pl.ds(obase, ROWS_PER_W)])


def _run_sc_ball_gather(xyzb_flat, newb_flat, payload_flat):
    mesh = plsc.VectorSubcoreMesh(core_axis_name="c", subcore_axis_name="s")
    k = pl.kernel(
        _sc_body,
        out_type=jax.ShapeDtypeStruct((NSAMPLE, 6, B * NPOINT), jnp.float32),
        mesh=mesh,
        compiler_params=pltpu.CompilerParams(needs_layout_passes=False),
        scratch_types=[
            pltpu.VMEM((4 * N,), jnp.float32),
            pltpu.VMEM((8 * ROWS_PER_W,), jnp.float32),
            pltpu.VMEM((6 * N,), jnp.float32),
            pltpu.VMEM((4 * (NSAMPLE + L),), jnp.int32),
            pltpu.VMEM((6 * PLANE_W,), jnp.float32),
        ],
    )
    return k(xyzb_flat, newb_flat, payload_flat)


R_TOTAL = B * NPOINT * NSAMPLE
BS = B * NPOINT
CT = 8192
NCT = BS // CT


def _bn_relu(y, stats, g, bt):
    m = stats[:, 0:1] / R_TOTAL
    v = stats[:, 1:2] / R_TOTAL - m * m
    return jnp.maximum(g * (y - m) * lax.rsqrt(v + EPS) + bt, 0.0)


def _mlp_chain(w_refs, stat_refs, x, nlayers):
    y = x
    for li in range(nlayers):
        if li > 0:
            g, bt = w_refs[4 * li - 2], w_refs[4 * li - 1]
            y = _bn_relu(y, stat_refs[li - 1][...], g[...], bt[...])
        W, b = w_refs[4 * li], w_refs[4 * li + 1]
        y = lax.dot_general(W[...], y, (((1,), (0,)), ((), ())),
                            preferred_element_type=jnp.float32) + b[...]
    return y


def _make_stats_kernel(nlayer, nw, ns):
    def kern(*refs):
        w_refs = refs[:nw]
        stat_refs = refs[nw:nw + ns]
        x_ref, out_ref, acc_ref = refs[nw + ns:]
        i = pl.program_id(0)
        y = _mlp_chain(w_refs, stat_refs, x_ref[0], nlayer + 1)

        @pl.when(i == 0)
        def _():
            acc_ref[...] = jnp.zeros_like(acc_ref)

        acc_ref[:, 0:1] += jnp.sum(y, axis=1, keepdims=True)
        acc_ref[:, 1:2] += jnp.sum(y * y, axis=1, keepdims=True)

        @pl.when(i == NSAMPLE - 1)
        def _():
            out_ref[...] = acc_ref[...]

    return kern


def _final_kernel(*refs):
    w_refs = refs[:12]
    stat_refs = refs[12:15]
    x_ref, out_ref, acc_ref = refs[15:]
    k = pl.program_id(1)
    y = _mlp_chain(w_refs, stat_refs, x_ref[0], 3)
    y = _bn_relu(y, stat_refs[2][...], w_refs[10][...], w_refs[11][...])

    @pl.when(k == 0)
    def _():
        acc_ref[...] = y

    @pl.when(k > 0)
    def _():
        acc_ref[...] = jnp.maximum(acc_ref[...], y)

    @pl.when(k == NSAMPLE - 1)
    def _():
        out_ref[...] = acc_ref[...]


def _full_spec(a):
    nd = a.ndim
    return pl.BlockSpec(a.shape, lambda i, _nd=nd: (0,) * _nd)


def _run_mlp(xplanes, params, interpret=False):
    w_args = []
    for (W, bias, g, bt) in params:
        w_args += [W, bias, g, bt]
    w_specs = [_full_spec(a) for a in w_args]

    stats = []
    for li in range(3):
        oc = params[li][0].shape[0]
        nw = 4 * li + 2
        st = pl.pallas_call(
            _make_stats_kernel(li, nw, li),
            grid=(NSAMPLE,),
            in_specs=w_specs[:nw]
            + [_full_spec(s) for s in stats]
            + [pl.BlockSpec((1, 6, BS), lambda i: (i, 0, 0))],
            out_specs=pl.BlockSpec((oc, 2), lambda i: (0, 0)),
            out_shape=jax.ShapeDtypeStruct((oc, 2), jnp.float32),
            scratch_shapes=[pltpu.VMEM((oc, 2), jnp.float32)],
            interpret=interpret,
        )(*w_args[:nw], *stats, xplanes)
        stats.append(st)

    out = pl.pallas_call(
        _final_kernel,
        grid=(NCT, NSAMPLE),
        in_specs=[pl.BlockSpec(s.block_shape, lambda i, k, _f=s.index_map: _f(i))
                  for s in w_specs]
        + [pl.BlockSpec(st.shape, lambda i, k, _nd=st.ndim: (0,) * _nd)
           for st in stats]
        + [pl.BlockSpec((1, 6, CT), lambda i, k: (k, 0, i))],
        out_specs=pl.BlockSpec((128, CT), lambda i, k: (0, i)),
        out_shape=jax.ShapeDtypeStruct((128, BS), jnp.float32),
        scratch_shapes=[pltpu.VMEM((128, CT), jnp.float32)],
        interpret=interpret,
    )(*w_args, *stats, xplanes)
    return out


def kernel(xyz, points, W0, b0, g0, beta0, W1, b1, g1, beta1, W2, b2, g2, beta2,
           interpret=False):
    new_xyz_t, xyzb, newb = _run_fps(xyz, interpret=interpret)
    xyzb_flat = xyzb.reshape(B, 4 * N)
    newb_flat = newb.reshape(B, 8, 2, NPOINT // 2).transpose(0, 2, 1, 3).reshape(
        B, 2, 8 * (NPOINT // 2))
    payload_flat = jnp.concatenate([xyz, points], axis=1).reshape(B, 6 * N)
    xplanes = _run_sc_ball_gather(xyzb_flat, newb_flat, payload_flat)
    params = [
        (W0, b0[:, None], g0[:, None], beta0[:, None]),
        (W1, b1[:, None], g1[:, None], beta1[:, None]),
        (W2, b2[:, None], g2[:, None], beta2[:, None]),
    ]
    pooled = _run_mlp(xplanes, params, interpret=interpret)
    new_points = pooled.reshape(128, B, NPOINT).transpose(1, 0, 2)
    return new_xyz_t, new_points


# speedup vs baseline: 3.9715x; 1.0004x over previous
import jax
import jax.numpy as jnp
from jax import lax
from jax.experimental import pallas as pl
from jax.experimental.pallas import tpu as pltpu
from jax.experimental.pallas import tpu_sc as plsc

NPOINT = 512
RADIUS = 0.2
NSAMPLE = 32
B = 16
N = 4096
EPS = 1e-5

NC, NS, L = 2, 16, 16
NW = NC * NS
ROWS_PER_W = B * NPOINT // NW


def _rtne_bf16(v):
    u = lax.bitcast_convert_type(v, jnp.uint32)
    u2 = u + jnp.uint32(0x7FFF) + ((u >> 16) & jnp.uint32(1))
    return lax.bitcast_convert_type(u2 & jnp.uint32(0xFFFF0000), jnp.float32)


def _fps_kernel(xyz_ref, out_ref, xyzb_ref, newb_ref, dist_ref):
    x = xyz_ref[:, 0, :]
    y = xyz_ref[:, 1, :]
    z = xyz_ref[:, 2, :]
    p48 = jnp.concatenate([x, y, z], axis=0)
    iota = lax.broadcasted_iota(jnp.int32, (B, N), 1)
    siota3 = lax.broadcasted_iota(jnp.int32, (3, B, NPOINT), 2)
    big = jnp.int32(N)

    m0 = jnp.max(x, axis=1, keepdims=True)
    far0 = jnp.min(jnp.where(x == m0, iota, big), axis=1, keepdims=True)
    dist_ref[...] = jnp.full((B, N), 1e10, dtype=jnp.float32)
    acc0 = jnp.zeros((3, B, NPOINT), jnp.float32)

    def body(i, carry):
        far, acc3 = carry
        onehot = iota == far
        oh3 = jnp.concatenate([onehot, onehot, onehot], axis=0)
        ext = jnp.sum(jnp.where(oh3, p48, 0.0), axis=1, keepdims=True)
        acc3 = jnp.where(siota3 == i, ext.reshape(3, B, 1), acc3)
        diff = p48 - ext
        sq = diff * diff
        d = (sq[0:B] + sq[B:2 * B]) + sq[2 * B:3 * B]
        dist = dist_ref[...]
        dist = jnp.where(d < dist, d, dist)
        dist_ref[...] = dist
        m = jnp.max(dist, axis=1, keepdims=True)
        far = jnp.min(jnp.where(dist == m, iota, big), axis=1, keepdims=True)
        return far, acc3

    _, acc3 = lax.fori_loop(0, NPOINT, body, (far0, acc0))
    out_ref[...] = jnp.stack([acc3[0], acc3[1], acc3[2]], axis=1)

    tn = (x * x + y * y) + z * z
    xyzb_ref[...] = jnp.stack([_rtne_bf16(x), _rtne_bf16(y), _rtne_bf16(z), tn],
                              axis=1)
    cx = acc3[0]
    cy = acc3[1]
    cz = acc3[2]
    sn = (cx * cx + cy * cy) + cz * cz
    newb_ref[...] = jnp.stack(
        [_rtne_bf16(cx), _rtne_bf16(cy), _rtne_bf16(cz), sn, cx, cy, cz,
         jnp.zeros_like(cx)], axis=1)


def _run_fps(xyz, interpret=False):
    return pl.pallas_call(
        _fps_kernel,
        out_shape=(
            jax.ShapeDtypeStruct((B, 3, NPOINT), jnp.float32),
            jax.ShapeDtypeStruct((B, 4, N), jnp.float32),
            jax.ShapeDtypeStruct((B, 8, NPOINT), jnp.float32),
        ),
        scratch_shapes=[pltpu.VMEM((B, N), jnp.float32)],
        interpret=interpret,
    )(xyz)


R2 = RADIUS ** 2
NCHUNK = N // L
PLANE_W = ROWS_PER_W * NSAMPLE


def _sc_body(xyzb_hbm, newb_hbm, payload_hbm, out_hbm,
             xyzb_v, newb_v, payload_v, rowbuf, plane_v):
    cid = lax.axis_index("c")
    sid = lax.axis_index("s")
    wid = sid * NC + cid
    b = wid // 2
    half = wid % 2

    pltpu.sync_copy(xyzb_hbm.at[b], xyzb_v)
    pltpu.sync_copy(newb_hbm.at[b, half], newb_v)
    pltpu.sync_copy(payload_hbm.at[b], payload_v)

    iota16 = lax.iota(jnp.int32, 16)
    KSTRIDE = 6 * ROWS_PER_W
    kdest0 = iota16 * KSTRIDE
    kdest1 = (iota16 + L) * KSTRIDE
    UB = 8
    RB = NSAMPLE + L

    NLANES_R = 4
    HALF_R = ROWS_PER_W // NLANES_R

    def pair_body(r, dummy):
        rows = tuple(r + t * HALF_R for t in range(NLANES_R))

        def bc(ch, rr):
            return plsc.load_gather(
                newb_v, [jnp.full((L,), ch * ROWS_PER_W, jnp.int32) + rr])

        sv = [tuple(bc(ch, rr) for ch in range(7)) for rr in rows]

        def cond(carry):
            c = carry[0]
            cnts = carry[1:1 + NLANES_R]
            active = cnts[0] < NSAMPLE
            for t in range(1, NLANES_R):
                active = active | (cnts[t] < NSAMPLE)
            return (c < NCHUNK) & active

        def body(carry):
            c = carry[0]
            cntvs = list(carry[1 + NLANES_R:])
            for u in range(UB):
                base = (c + u) * L
                px = xyzb_v[pl.ds(base, L)]
                py = xyzb_v[pl.ds(N + base, L)]
                pz = xyzb_v[pl.ds(2 * N + base, L)]
                tn = xyzb_v[pl.ds(3 * N + base, L)]
                idxv = base + iota16
                for t in range(NLANES_R):
                    sxb, syb, szb, snv = sv[t][:4]
                    dot = sxb * px + syb * py
                    dot = dot + szb * pz
                    dd = (jnp.float32(-2.0) * dot + snv) + tn
                    mask = dd <= R2
                    incl = plsc.cumsum(mask.astype(jnp.int32))
                    dest = (cntvs[t] + incl) - 1
                    wmask = mask & (dest < RB)
                    plsc.store_scatter(rowbuf, [dest + t * RB], idxv,
                                       mask=wmask)
                    cntvs[t] = cntvs[t] + plsc.all_reduce_population_count(mask)
            return ((c + UB,) + tuple(jnp.max(cv) for cv in cntvs)
                    + tuple(cntvs))

        zero = jnp.zeros((L,), jnp.int32)
        fin = lax.while_loop(
            cond, body,
            (jnp.int32(0),) + (jnp.int32(0),) * NLANES_R + (zero,) * NLANES_R)
        cntvs_fin = fin[1 + NLANES_R:]

        for t, rr, cntv in [(t, rows[t], cntvs_fin[t]) for t in range(NLANES_R)]:
            sx, sy, sz = sv[t][4:7]
            off = t * RB
            first = plsc.load_gather(rowbuf, [jnp.full((L,), off, jnp.int32)])
            v0 = jnp.where(iota16 < cntv, rowbuf[pl.ds(off, L)], first)
            v1 = jnp.where((iota16 + L) < cntv, rowbuf[pl.ds(off + L, L)],
                           first)

            for c in range(6):
                sub = (sx, sy, sz, None, None, None)[c]
                cbase = c * ROWS_PER_W + rr
                for vv, kd in ((v0, kdest0), (v1, kdest1)):
                    val = plsc.load_gather(payload_v, [vv + (c * N)])
                    if sub is not None:
                        val = val - sub
                    plsc.store_scatter(plane_v, [kd + cbase], val)
        return dummy

    lax.fori_loop(0, HALF_R, pair_body, jnp.int32(0))

    obase = wid * ROWS_PER_W
    for k in range(NSAMPLE):
        for c in range(6):
            pltpu.sync_copy(
                plane_v.at[pl.ds((k * 6 + c) * ROWS_PER_W, ROWS_PER_W)],
                out_hbm.at[k, c, pl.ds(obase, ROWS_PER_W)])


def _run_sc_ball_gather(xyzb_flat, newb_flat, payload_flat):
    mesh = plsc.VectorSubcoreMesh(core_axis_name="c", subcore_axis_name="s")
    k = pl.kernel(
        _sc_body,
        out_type=jax.ShapeDtypeStruct((NSAMPLE, 6, B * NPOINT), jnp.float32),
        mesh=mesh,
        compiler_params=pltpu.CompilerParams(needs_layout_passes=False),
        scratch_types=[
            pltpu.VMEM((4 * N,), jnp.float32),
            pltpu.VMEM((8 * ROWS_PER_W,), jnp.float32),
            pltpu.VMEM((6 * N,), jnp.float32),
            pltpu.VMEM((4 * (NSAMPLE + L),), jnp.int32),
            pltpu.VMEM((6 * PLANE_W,), jnp.float32),
        ],
    )
    return k(xyzb_flat, newb_flat, payload_flat)


R_TOTAL = B * NPOINT * NSAMPLE
BS = B * NPOINT
CT = 8192
NCT = BS // CT


def _bn_relu(y, stats, g, bt):
    m = stats[:, 0:1] / R_TOTAL
    v = stats[:, 1:2] / R_TOTAL - m * m
    return jnp.maximum(g * (y - m) * lax.rsqrt(v + EPS) + bt, 0.0)


def _mlp_chain(w_refs, stat_refs, x, nlayers):
    y = x
    for li in range(nlayers):
        if li > 0:
            g, bt = w_refs[4 * li - 2], w_refs[4 * li - 1]
            y = _bn_relu(y, stat_refs[li - 1][...], g[...], bt[...])
        W, b = w_refs[4 * li], w_refs[4 * li + 1]
        y = lax.dot_general(W[...], y, (((1,), (0,)), ((), ())),
                            preferred_element_type=jnp.float32) + b[...]
    return y


def _make_stats_kernel(nlayer, nw, ns):
    def kern(*refs):
        w_refs = refs[:nw]
        stat_refs = refs[nw:nw + ns]
        x_ref, out_ref, acc_ref = refs[nw + ns:]
        i = pl.program_id(0)
        y = _mlp_chain(w_refs, stat_refs, x_ref[0], nlayer + 1)

        @pl.when(i == 0)
        def _():
            acc_ref[...] = jnp.zeros_like(acc_ref)

        acc_ref[:, 0:1] += jnp.sum(y, axis=1, keepdims=True)
        acc_ref[:, 1:2] += jnp.sum(y * y, axis=1, keepdims=True)

        @pl.when(i == NSAMPLE - 1)
        def _():
            out_ref[...] = acc_ref[...]

    return kern


def _final_kernel(*refs):
    w_refs = refs[:12]
    stat_refs = refs[12:15]
    x_ref, out_ref, acc_ref = refs[15:]
    k = pl.program_id(1)
    y = _mlp_chain(w_refs, stat_refs, x_ref[0], 3)
    y = _bn_relu(y, stat_refs[2][...], w_refs[10][...], w_refs[11][...])

    @pl.when(k == 0)
    def _():
        acc_ref[...] = y

    @pl.when(k > 0)
    def _():
        acc_ref[...] = jnp.maximum(acc_ref[...], y)

    @pl.when(k == NSAMPLE - 1)
    def _():
        out_ref[...] = acc_ref[...]


def _full_spec(a):
    nd = a.ndim
    return pl.BlockSpec(a.shape, lambda i, _nd=nd: (0,) * _nd)


def _run_mlp(xplanes, params, interpret=False):
    w_args = []
    for (W, bias, g, bt) in params:
        w_args += [W, bias, g, bt]
    w_specs = [_full_spec(a) for a in w_args]

    stats = []
    for li in range(3):
        oc = params[li][0].shape[0]
        nw = 4 * li + 2
        st = pl.pallas_call(
            _make_stats_kernel(li, nw, li),
            grid=(NSAMPLE,),
            in_specs=w_specs[:nw]
            + [_full_spec(s) for s in stats]
            + [pl.BlockSpec((1, 6, BS), lambda i: (i, 0, 0))],
            out_specs=pl.BlockSpec((oc, 2), lambda i: (0, 0)),
            out_shape=jax.ShapeDtypeStruct((oc, 2), jnp.float32),
            scratch_shapes=[pltpu.VMEM((oc, 2), jnp.float32)],
            interpret=interpret,
        )(*w_args[:nw], *stats, xplanes)
        stats.append(st)

    out = pl.pallas_call(
        _final_kernel,
        grid=(NCT, NSAMPLE),
        in_specs=[pl.BlockSpec(s.block_shape, lambda i, k, _f=s.index_map: _f(i))
                  for s in w_specs]
        + [pl.BlockSpec(st.shape, lambda i, k, _nd=st.ndim: (0,) * _nd)
           for st in stats]
        + [pl.BlockSpec((1, 6, CT), lambda i, k: (k, 0, i))],
        out_specs=pl.BlockSpec((128, CT), lambda i, k: (0, i)),
        out_shape=jax.ShapeDtypeStruct((128, BS), jnp.float32),
        scratch_shapes=[pltpu.VMEM((128, CT), jnp.float32)],
        interpret=interpret,
    )(*w_args, *stats, xplanes)
    return out


def kernel(xyz, points, W0, b0, g0, beta0, W1, b1, g1, beta1, W2, b2, g2, beta2,
           interpret=False):
    new_xyz_t, xyzb, newb = _run_fps(xyz, interpret=interpret)
    xyzb_flat = xyzb.reshape(B, 4 * N)
    newb_flat = newb.reshape(B, 8, 2, NPOINT // 2).transpose(0, 2, 1, 3).reshape(
        B, 2, 8 * (NPOINT // 2))
    payload_flat = jnp.concatenate([xyz, points], axis=1).reshape(B, 6 * N)
    xplanes = _run_sc_ball_gather(xyzb_flat, newb_flat, payload_flat)
    params = [
        (W0, b0[:, None], g0[:, None], beta0[:, None]),
        (W1, b1[:, None], g1[:, None], beta1[:, None]),
        (W2, b2[:, None], g2[:, None], beta2[:, None]),
    ]
    pooled = _run_mlp(xplanes, params, interpret=interpret)
    new_points = pooled.reshape(128, B, NPOINT).transpose(1, 0, 2)
    return new_xyz_t, new_points
